# Initial kernel scaffold; baseline (speedup 1.0000x reference)
import jax
import jax.numpy as jnp
from jax.experimental import pallas as pl


def kernel(emb, weight1, root1, bias1, weight2, root2, bias2, edge_index, edge_type):
    raise NotImplementedError("write your pallas kernel here")


# speedup vs baseline: 12.9706x; 12.9706x over previous
import functools

import jax
import jax.numpy as jnp
from jax import lax
from jax.experimental import pallas as pl
from jax.experimental.pallas import tpu as pltpu
from jax.experimental.pallas import tpu_sc as plsc

N = 50000
E = 800000
R = 8
D = 48
H = 48
L = 16

NTILES = 32
CHUNKS = 25
CHUNK = 1024
EP = NTILES * CHUNKS * CHUNK
EROWS = EP // 128
NK = 401408
NKROWS = NK // 128
NKT = NK // 16
NA = 50048
AROWS = NA // 16


def _xw1_body(emb_ref, w0_ref, w1_ref, w2_ref, o0_ref, o1_ref, o2_ref):
    x = emb_ref[...]
    o0_ref[...] = jnp.dot(x, w0_ref[...], preferred_element_type=jnp.float32)
    o1_ref[...] = jnp.dot(x, w1_ref[...], preferred_element_type=jnp.float32)
    o2_ref[...] = jnp.dot(x, w2_ref[...], preferred_element_type=jnp.float32)


def _inv_body(hist_ref, inv_ref):
    cnt = hist_ref[0] + hist_ref[1]
    row = lax.broadcasted_iota(jnp.int32, (NKROWS, 128), 0)
    inv_ref[...] = jnp.where(row < (N * R) // 128,
                             1.0 / jnp.maximum(cnt, 1.0), 0.0)


def _h1_body(a0_ref, a1_ref, a2_ref, emb_ref, r10_ref, r11_ref, r12_ref,
             b1_ref, w20_ref, w21_ref, w22_ref,
             h0_ref, h1_ref, h2_ref, xw2_ref):
    x = emb_ref[...]
    hs = []
    for k, (a_ref, r_ref) in enumerate(
            [(a0_ref, r10_ref), (a1_ref, r11_ref), (a2_ref, r12_ref)]):
        h = a_ref[0] + a_ref[1] + jnp.dot(
            x, r_ref[...], preferred_element_type=jnp.float32
        ) + b1_ref[0:1, k * 16:(k + 1) * 16]
        hs.append(jnp.maximum(h, 0.0))
    h0_ref[...] = hs[0]
    h1_ref[...] = hs[1]
    h2_ref[...] = hs[2]
    xw2_ref[...] = (
        jnp.dot(hs[0], w20_ref[...], preferred_element_type=jnp.float32)
        + jnp.dot(hs[1], w21_ref[...], preferred_element_type=jnp.float32)
        + jnp.dot(hs[2], w22_ref[...], preferred_element_type=jnp.float32))


def _out_body(agg2_ref, h0_ref, h1_ref, h2_ref, r20_ref, r21_ref, r22_ref,
              b2_ref, out_ref):
    y = (agg2_ref[0] + agg2_ref[1] + b2_ref[0:1, :]
         + jnp.dot(h0_ref[...], r20_ref[...],
                   preferred_element_type=jnp.float32)
         + jnp.dot(h1_ref[...], r21_ref[...],
                   preferred_element_type=jnp.float32)
         + jnp.dot(h2_ref[...], r22_ref[...],
                   preferred_element_type=jnp.float32))
    out_ref[...] = jax.nn.sigmoid(y)


def _sc_mesh():
    return plsc.VectorSubcoreMesh(core_axis_name="c", subcore_axis_name="s")


def _make_hist():
    mesh = _sc_mesh()

    @functools.partial(
        pl.kernel,
        out_type=jax.ShapeDtypeStruct((2 * NK,), jnp.float32),
        mesh=mesh,
        scratch_types=[
            pltpu.VMEM((8, 3, 128), jnp.int32),
            pltpu.VMEM((8, 128), jnp.int32),
            pltpu.VMEM((128,), jnp.float32),
            pltpu.VMEM((4096,), jnp.float32),
            pltpu.VMEM_SHARED((NK,), jnp.float32),
        ],
    )
    def hist(edges_hbm, out_hbm, ebuf, kidx, ones_v, zbuf, hist_sh):
        cid = lax.axis_index("c")
        sid = lax.axis_index("s")
        wid = cid * 16 + sid

        def zfill(i, _):
            zbuf[pl.ds(i * 16, 16)] = jnp.zeros((16,), jnp.float32)
            return _
        lax.fori_loop(0, 256, zfill, None)

        def ofill(i, _):
            ones_v[pl.ds(i * 16, 16)] = jnp.full((16,), 1.0, jnp.float32)
            return _
        lax.fori_loop(0, 8, ofill, None)

        zslot = sid * NKT
        for b in range(6):
            pltpu.sync_copy(zbuf.at[:],
                            hist_sh.at[pl.ds(zslot + b * 4096, 4096)])
        pltpu.sync_copy(zbuf.at[pl.ds(0, 512)],
                        hist_sh.at[pl.ds(zslot + 6 * 4096, 512)])
        plsc.subcore_barrier()

        def chunk_body(c, carry):
            base = (wid * CHUNKS + c) * 8
            pltpu.sync_copy(edges_hbm.at[pl.ds(base, 8)], ebuf)
            for j in range(8):
                for g in range(8):
                    s = pl.ds(g * 16, 16)
                    kidx[j, s] = ebuf[j, 1, s] * R + ebuf[j, 2, s]
            for j in range(8):
                pltpu.sync_copy(ones_v, hist_sh.at[kidx.at[j]], add=True)
            return carry
        lax.fori_loop(0, CHUNKS, chunk_body, None)

        plsc.subcore_barrier()
        off = pl.multiple_of(cid * NK + sid * NKT, 128)
        pltpu.sync_copy(hist_sh.at[pl.ds(sid * NKT, NKT)],
                        out_hbm.at[pl.ds(off, NKT)])

    return hist


def _make_norm():
    mesh = _sc_mesh()

    @functools.partial(
        pl.kernel,
        out_type=jax.ShapeDtypeStruct((EROWS, 128), jnp.float32),
        mesh=mesh,
        scratch_types=[
            pltpu.VMEM((8, 3, 128), jnp.int32),
            pltpu.VMEM((8, 128), jnp.int32),
            pltpu.VMEM((8, 128), jnp.float32),
            pltpu.SemaphoreType.DMA,
        ],
    )
    def norm(edges_hbm, inv_hbm, out_hbm, ebuf, kidx, nbuf, sem):
        cid = lax.axis_index("c")
        sid = lax.axis_index("s")
        wid = cid * 16 + sid

        def chunk_body(c, carry):
            base = (wid * CHUNKS + c) * 8
            pltpu.sync_copy(edges_hbm.at[pl.ds(base, 8)], ebuf)
            for j in range(8):
                for g in range(8):
                    s = pl.ds(g * 16, 16)
                    kidx[j, s] = ebuf[j, 1, s] * R + ebuf[j, 2, s]
            for j in range(8):
                pltpu.async_copy(inv_hbm.at[kidx.at[j]], nbuf.at[j],
                                 sem).wait()
            pltpu.sync_copy(nbuf, out_hbm.at[pl.ds(base, 8)])
            return carry
        lax.fori_loop(0, CHUNKS, chunk_body, None)

    return norm


def _make_edge(W):
    mesh = _sc_mesh()
    nh = W // 16

    @functools.partial(
        pl.kernel,
        out_type=jax.ShapeDtypeStruct((2, NA, W), jnp.float32),
        mesh=mesh,
        scratch_types=[
            pltpu.VMEM((8, 3, 128), jnp.int32),
            pltpu.VMEM((8, 128), jnp.int32),
            pltpu.VMEM((8, 128), jnp.int32),
            pltpu.VMEM((8, 128), jnp.float32),
            pltpu.VMEM((8, 128, W), jnp.float32),
            pltpu.VMEM((128, W), jnp.float32),
            pltpu.VMEM_SHARED((NA, W), jnp.float32),
            pltpu.SemaphoreType.DMA,
        ],
        compiler_params=pltpu.CompilerParams(use_tc_tiling_on_sc=False),
    )
    def edge(edges_hbm, norm_hbm, table_hbm, out_hbm,
             ebuf, sidx, didx, nbuf, rows, zbuf, agg_sh, sem):
        cid = lax.axis_index("c")
        sid = lax.axis_index("s")
        wid = cid * 16 + sid

        def zfill(i, carry):
            for h in range(nh):
                zbuf[i, pl.ds(h * 16, 16)] = jnp.zeros((16,), jnp.float32)
            return carry
        lax.fori_loop(0, 128, zfill, None)

        r0 = sid * AROWS
        for b in range(24):
            pltpu.sync_copy(zbuf.at[:], agg_sh.at[pl.ds(r0 + b * 128, 128)])
        pltpu.sync_copy(zbuf.at[pl.ds(0, 56)],
                        agg_sh.at[pl.ds(r0 + 24 * 128, 56)])
        plsc.subcore_barrier()

        def chunk_body(c, carry):
            base = (wid * CHUNKS + c) * 8
            pltpu.sync_copy(edges_hbm.at[pl.ds(base, 8)], ebuf)
            pltpu.sync_copy(norm_hbm.at[pl.ds(base, 8)], nbuf)
            for j in range(8):
                for g in range(8):
                    s = pl.ds(g * 16, 16)
                    sidx[j, s] = ebuf[j, 0, s] * R + ebuf[j, 2, s]
                    didx[j, s] = ebuf[j, 1, s]
            for j in range(8):
                pltpu.async_copy(table_hbm.at[sidx.at[j]], rows.at[j],
                                 sem).wait()
            for j in range(8):
                def scale(g, carry2):
                    nv = nbuf[j, pl.ds(g * 16, 16)]
                    for i in range(16):
                        sc = nv[i]
                        e = g * 16 + i
                        for h in range(nh):
                            sl = pl.ds(h * 16, 16)
                            rows[j, e, sl] = rows[j, e, sl] * sc
                    return carry2
                lax.fori_loop(0, 8, scale, None)
            for j in range(8):
                pltpu.sync_copy(rows.at[j], agg_sh.at[didx.at[j]], add=True)
            return carry
        lax.fori_loop(0, CHUNKS, chunk_body, None)

        plsc.subcore_barrier()
        roff = pl.multiple_of(sid * AROWS, 8)
        pltpu.sync_copy(agg_sh.at[pl.ds(roff, AROWS)],
                        out_hbm.at[cid, pl.ds(roff, AROWS)])

    return edge


def _tc_call(body, grid, in_specs, out_specs, out_shape):
    return pl.pallas_call(body, grid=grid, in_specs=in_specs,
                          out_specs=out_specs, out_shape=out_shape)


def kernel(emb, weight1, root1, bias1, weight2, root2, bias2,
           edge_index, edge_type):
    f32 = jnp.float32
    src = edge_index[0]
    dst = edge_index[1]
    pad = EP - E
    src_p = jnp.concatenate([src, jnp.zeros((pad,), jnp.int32)])
    dst_p = jnp.concatenate([dst, jnp.full((pad,), N, jnp.int32)])
    typ_p = jnp.concatenate([edge_type, jnp.zeros((pad,), jnp.int32)])
    edges3 = jnp.stack([src_p.reshape(EROWS, 128),
                        dst_p.reshape(EROWS, 128),
                        typ_p.reshape(EROWS, 128)], axis=1)

    w1s = [weight1[:, :, k * 16:(k + 1) * 16].transpose(1, 0, 2)
           .reshape(D, R * 16) for k in range(3)]
    w2m = weight2.transpose(1, 0, 2).reshape(H, R * L)
    b1t = jnp.tile(bias1.reshape(1, H), (8, 1))
    b2t = jnp.tile(bias2.reshape(1, L), (8, 1))

    BN = 1000
    GRID = N // BN

    full = lambda shp: pl.BlockSpec(shp, lambda i: (0,) * len(shp))
    rowblk = lambda w: pl.BlockSpec((BN, w), lambda i: (i, 0))
    aggblk = lambda w: pl.BlockSpec((2, BN, w), lambda i: (0, i, 0))

    xws = _tc_call(
        _xw1_body, (GRID,),
        [rowblk(D)] + [full((D, R * 16))] * 3,
        [rowblk(R * 16)] * 3,
        [jax.ShapeDtypeStruct((N, R * 16), f32)] * 3,
    )(emb, *w1s)
    xws = [x.reshape(N * R, 16) for x in xws]

    hist = _make_hist()(edges3)

    inv = _tc_call(
        _inv_body, (1,),
        [pl.BlockSpec((2, NKROWS, 128), lambda i: (0, 0, 0))],
        pl.BlockSpec((NKROWS, 128), lambda i: (0, 0)),
        jax.ShapeDtypeStruct((NKROWS, 128), f32),
    )(hist.reshape(2, NKROWS, 128)).reshape(NK)

    norm2d = _make_norm()(edges3, inv)

    edge16 = _make_edge(16)
    aggs = [edge16(edges3, norm2d, x) for x in xws]

    h0, h1, h2, xw2 = _tc_call(
        _h1_body, (GRID,),
        [aggblk(16)] * 3 + [rowblk(D)] + [full((D, 16))] * 3
        + [full((8, H))] + [full((16, R * L))] * 3,
        [rowblk(16)] * 3 + [rowblk(R * L)],
        [jax.ShapeDtypeStruct((N, 16), f32)] * 3
        + [jax.ShapeDtypeStruct((N, R * L), f32)],
    )(*[a[:, :N, :] for a in aggs], emb,
      *[root1[:, k * 16:(k + 1) * 16] for k in range(3)], b1t,
      *[w2m[k * 16:(k + 1) * 16, :] for k in range(3)])

    agg2 = edge16(edges3, norm2d, xw2.reshape(N * R, L))

    out = _tc_call(
        _out_body, (GRID,),
        [aggblk(16), rowblk(16), rowblk(16), rowblk(16),
         full((16, L)), full((16, L)), full((16, L)), full((8, L))],
        rowblk(L),
        jax.ShapeDtypeStruct((N, L), f32),
    )(agg2[:, :N, :], h0, h1, h2,
      *[root2[k * 16:(k + 1) * 16, :] for k in range(3)], b2t)

    return out


# speedup vs baseline: 17.6979x; 1.3645x over previous
import functools

import jax
import jax.numpy as jnp
from jax import lax
from jax.experimental import pallas as pl
from jax.experimental.pallas import tpu as pltpu
from jax.experimental.pallas import tpu_sc as plsc

N = 50000
E = 800000
R = 8
D = 48
H = 48
L = 16

NTILES = 32
CHUNKS = 25
CHUNK = 1024
EP = NTILES * CHUNKS * CHUNK
EROWS = EP // 128
NK = 401408
NKROWS = NK // 128
NKT = NK // 16
NA = 50048
AROWS = NA // 16


def _xw1_body(emb_ref, w0_ref, w1_ref, w2_ref, o0_ref, o1_ref, o2_ref):
    x = emb_ref[...]
    o0_ref[...] = jnp.dot(x, w0_ref[...], preferred_element_type=jnp.float32)
    o1_ref[...] = jnp.dot(x, w1_ref[...], preferred_element_type=jnp.float32)
    o2_ref[...] = jnp.dot(x, w2_ref[...], preferred_element_type=jnp.float32)


def _inv_body(hist_ref, inv_ref):
    cnt = hist_ref[0] + hist_ref[1]
    row = lax.broadcasted_iota(jnp.int32, (NKROWS, 128), 0)
    inv_ref[...] = jnp.where(row < (N * R) // 128,
                             1.0 / jnp.maximum(cnt, 1.0), 0.0)


def _h1_body(a0_ref, a1_ref, a2_ref, emb_ref, r10_ref, r11_ref, r12_ref,
             b1_ref, w20_ref, w21_ref, w22_ref,
             h0_ref, h1_ref, h2_ref, xw2_ref):
    x = emb_ref[...]
    hs = []
    for k, (a_ref, r_ref) in enumerate(
            [(a0_ref, r10_ref), (a1_ref, r11_ref), (a2_ref, r12_ref)]):
        h = a_ref[0] + a_ref[1] + jnp.dot(
            x, r_ref[...], preferred_element_type=jnp.float32
        ) + b1_ref[0:1, k * 16:(k + 1) * 16]
        hs.append(jnp.maximum(h, 0.0))
    h0_ref[...] = hs[0]
    h1_ref[...] = hs[1]
    h2_ref[...] = hs[2]
    xw2_ref[...] = (
        jnp.dot(hs[0], w20_ref[...], preferred_element_type=jnp.float32)
        + jnp.dot(hs[1], w21_ref[...], preferred_element_type=jnp.float32)
        + jnp.dot(hs[2], w22_ref[...], preferred_element_type=jnp.float32))


def _out_body(agg2_ref, h0_ref, h1_ref, h2_ref, r20_ref, r21_ref, r22_ref,
              b2_ref, out_ref):
    y = (agg2_ref[0] + agg2_ref[1] + b2_ref[0:1, :]
         + jnp.dot(h0_ref[...], r20_ref[...],
                   preferred_element_type=jnp.float32)
         + jnp.dot(h1_ref[...], r21_ref[...],
                   preferred_element_type=jnp.float32)
         + jnp.dot(h2_ref[...], r22_ref[...],
                   preferred_element_type=jnp.float32))
    out_ref[...] = jax.nn.sigmoid(y)


def _sc_mesh():
    return plsc.VectorSubcoreMesh(core_axis_name="c", subcore_axis_name="s")


def _make_hist():
    mesh = _sc_mesh()

    @functools.partial(
        pl.kernel,
        out_type=jax.ShapeDtypeStruct((2 * NK,), jnp.float32),
        mesh=mesh,
        scratch_types=[
            pltpu.VMEM((8, 3, 128), jnp.int32),
            pltpu.VMEM((8, 128), jnp.int32),
            pltpu.VMEM((128,), jnp.float32),
            pltpu.VMEM((4096,), jnp.float32),
            pltpu.VMEM_SHARED((NK,), jnp.float32),
            pltpu.SemaphoreType.DMA,
            pltpu.SemaphoreType.DMA,
        ],
    )
    def hist(edges_hbm, out_hbm, ebuf, kidx, ones_v, zbuf, hist_sh,
             sem_l, sem_s):
        cid = lax.axis_index("c")
        sid = lax.axis_index("s")
        wid = cid * 16 + sid

        def zfill(i, _):
            zbuf[pl.ds(i * 16, 16)] = jnp.zeros((16,), jnp.float32)
            return _
        lax.fori_loop(0, 256, zfill, None)

        def ofill(i, _):
            ones_v[pl.ds(i * 16, 16)] = jnp.full((16,), 1.0, jnp.float32)
            return _
        lax.fori_loop(0, 8, ofill, None)

        zslot = sid * NKT
        for b in range(6):
            pltpu.sync_copy(zbuf.at[:],
                            hist_sh.at[pl.ds(zslot + b * 4096, 4096)])
        pltpu.sync_copy(zbuf.at[pl.ds(0, 512)],
                        hist_sh.at[pl.ds(zslot + 6 * 4096, 512)])
        plsc.subcore_barrier()

        def chunk_body(c, carry):
            base = (wid * CHUNKS + c) * 8
            pltpu.async_copy(edges_hbm.at[pl.ds(base, 8)], ebuf, sem_l).wait()
            for j in range(8):
                for g in range(8):
                    s = pl.ds(g * 16, 16)
                    kidx[j, s] = ebuf[j, 1, s] * R + ebuf[j, 2, s]
            adds = [pltpu.async_copy(ones_v, hist_sh.at[kidx.at[j]], sem_s,
                                     add=True) for j in range(8)]
            for a in adds:
                a.wait()
            return carry
        lax.fori_loop(0, CHUNKS, chunk_body, None)

        plsc.subcore_barrier()
        off = pl.multiple_of(cid * NK + sid * NKT, 128)
        pltpu.sync_copy(hist_sh.at[pl.ds(sid * NKT, NKT)],
                        out_hbm.at[pl.ds(off, NKT)])

    return hist


def _make_norm():
    mesh = _sc_mesh()

    @functools.partial(
        pl.kernel,
        out_type=jax.ShapeDtypeStruct((EROWS, 128), jnp.float32),
        mesh=mesh,
        scratch_types=[
            pltpu.VMEM((8, 3, 128), jnp.int32),
            pltpu.VMEM((8, 128), jnp.int32),
            pltpu.VMEM((8, 128), jnp.float32),
            pltpu.SemaphoreType.DMA,
        ],
    )
    def norm(edges_hbm, inv_hbm, out_hbm, ebuf, kidx, nbuf, sem):
        cid = lax.axis_index("c")
        sid = lax.axis_index("s")
        wid = cid * 16 + sid

        def chunk_body(c, carry):
            base = (wid * CHUNKS + c) * 8
            pltpu.async_copy(edges_hbm.at[pl.ds(base, 8)], ebuf, sem).wait()
            for j in range(8):
                for g in range(8):
                    s = pl.ds(g * 16, 16)
                    kidx[j, s] = ebuf[j, 1, s] * R + ebuf[j, 2, s]
            gs = [pltpu.async_copy(inv_hbm.at[kidx.at[j]], nbuf.at[j], sem)
                  for j in range(8)]
            for g_ in gs:
                g_.wait()
            pltpu.sync_copy(nbuf, out_hbm.at[pl.ds(base, 8)])
            return carry
        lax.fori_loop(0, CHUNKS, chunk_body, None)

    return norm


def _make_edge(W):
    mesh = _sc_mesh()
    nh = W // 16

    @functools.partial(
        pl.kernel,
        out_type=jax.ShapeDtypeStruct((2, NA, W), jnp.float32),
        mesh=mesh,
        scratch_types=[
            pltpu.VMEM((8, 3, 128), jnp.int32),
            pltpu.VMEM((8, 128), jnp.int32),
            pltpu.VMEM((8, 128), jnp.int32),
            pltpu.VMEM((8, 128), jnp.float32),
            pltpu.VMEM((8, 128, W), jnp.float32),
            pltpu.VMEM((128, W), jnp.float32),
            pltpu.VMEM_SHARED((NA, W), jnp.float32),
            pltpu.SemaphoreType.DMA,
            pltpu.SemaphoreType.DMA,
            pltpu.SemaphoreType.DMA,
        ],
        compiler_params=pltpu.CompilerParams(use_tc_tiling_on_sc=False),
    )
    def edge(edges_hbm, norm_hbm, table_hbm, out_hbm,
             ebuf, sidx, didx, nbuf, rows, zbuf, agg_sh,
             sem_l, sem_g, sem_s):
        cid = lax.axis_index("c")
        sid = lax.axis_index("s")
        wid = cid * 16 + sid

        def zfill(i, carry):
            for h in range(nh):
                zbuf[i, pl.ds(h * 16, 16)] = jnp.zeros((16,), jnp.float32)
            return carry
        lax.fori_loop(0, 128, zfill, None)

        r0 = sid * AROWS
        for b in range(24):
            pltpu.sync_copy(zbuf.at[:], agg_sh.at[pl.ds(r0 + b * 128, 128)])
        pltpu.sync_copy(zbuf.at[pl.ds(0, 56)],
                        agg_sh.at[pl.ds(r0 + 24 * 128, 56)])
        plsc.subcore_barrier()

        def chunk_body(c, carry):
            base = (wid * CHUNKS + c) * 8
            le = pltpu.async_copy(edges_hbm.at[pl.ds(base, 8)], ebuf, sem_l)
            ln = pltpu.async_copy(norm_hbm.at[pl.ds(base, 8)], nbuf, sem_l)
            le.wait()
            ln.wait()
            for j in range(8):
                for g in range(8):
                    s = pl.ds(g * 16, 16)
                    sidx[j, s] = ebuf[j, 0, s] * R + ebuf[j, 2, s]
                    didx[j, s] = ebuf[j, 1, s]
            gs = [pltpu.async_copy(table_hbm.at[sidx.at[j]], rows.at[j],
                                   sem_g) for j in range(8)]
            adds = []
            for j in range(8):
                gs[j].wait()

                def scale(g, carry2):
                    nv = nbuf[j, pl.ds(g * 16, 16)]
                    for i in range(16):
                        sc = nv[i]
                        e = g * 16 + i
                        for h in range(nh):
                            sl = pl.ds(h * 16, 16)
                            rows[j, e, sl] = rows[j, e, sl] * sc
                    return carry2
                lax.fori_loop(0, 8, scale, None)
                adds.append(pltpu.async_copy(rows.at[j], agg_sh.at[didx.at[j]],
                                             sem_s, add=True))
            for a in adds:
                a.wait()
            return carry
        lax.fori_loop(0, CHUNKS, chunk_body, None)

        plsc.subcore_barrier()
        roff = pl.multiple_of(sid * AROWS, 8)
        pltpu.sync_copy(agg_sh.at[pl.ds(roff, AROWS)],
                        out_hbm.at[cid, pl.ds(roff, AROWS)])

    return edge


def _tc_call(body, grid, in_specs, out_specs, out_shape):
    return pl.pallas_call(body, grid=grid, in_specs=in_specs,
                          out_specs=out_specs, out_shape=out_shape)


def kernel(emb, weight1, root1, bias1, weight2, root2, bias2,
           edge_index, edge_type):
    f32 = jnp.float32
    src = edge_index[0]
    dst = edge_index[1]
    pad = EP - E
    src_p = jnp.concatenate([src, jnp.zeros((pad,), jnp.int32)])
    dst_p = jnp.concatenate([dst, jnp.full((pad,), N, jnp.int32)])
    typ_p = jnp.concatenate([edge_type, jnp.zeros((pad,), jnp.int32)])
    edges3 = jnp.stack([src_p.reshape(EROWS, 128),
                        dst_p.reshape(EROWS, 128),
                        typ_p.reshape(EROWS, 128)], axis=1)

    w1s = [weight1[:, :, k * 16:(k + 1) * 16].transpose(1, 0, 2)
           .reshape(D, R * 16) for k in range(3)]
    w2m = weight2.transpose(1, 0, 2).reshape(H, R * L)
    b1t = jnp.tile(bias1.reshape(1, H), (8, 1))
    b2t = jnp.tile(bias2.reshape(1, L), (8, 1))

    BN = 1000
    GRID = N // BN

    full = lambda shp: pl.BlockSpec(shp, lambda i: (0,) * len(shp))
    rowblk = lambda w: pl.BlockSpec((BN, w), lambda i: (i, 0))
    aggblk = lambda w: pl.BlockSpec((2, BN, w), lambda i: (0, i, 0))

    xws = _tc_call(
        _xw1_body, (GRID,),
        [rowblk(D)] + [full((D, R * 16))] * 3,
        [rowblk(R * 16)] * 3,
        [jax.ShapeDtypeStruct((N, R * 16), f32)] * 3,
    )(emb, *w1s)
    xws = [x.reshape(N * R, 16) for x in xws]

    hist = _make_hist()(edges3)

    inv = _tc_call(
        _inv_body, (1,),
        [pl.BlockSpec((2, NKROWS, 128), lambda i: (0, 0, 0))],
        pl.BlockSpec((NKROWS, 128), lambda i: (0, 0)),
        jax.ShapeDtypeStruct((NKROWS, 128), f32),
    )(hist.reshape(2, NKROWS, 128)).reshape(NK)

    norm2d = _make_norm()(edges3, inv)

    edge16 = _make_edge(16)
    aggs = [edge16(edges3, norm2d, x) for x in xws]

    h0, h1, h2, xw2 = _tc_call(
        _h1_body, (GRID,),
        [aggblk(16)] * 3 + [rowblk(D)] + [full((D, 16))] * 3
        + [full((8, H))] + [full((16, R * L))] * 3,
        [rowblk(16)] * 3 + [rowblk(R * L)],
        [jax.ShapeDtypeStruct((N, 16), f32)] * 3
        + [jax.ShapeDtypeStruct((N, R * L), f32)],
    )(*[a[:, :N, :] for a in aggs], emb,
      *[root1[:, k * 16:(k + 1) * 16] for k in range(3)], b1t,
      *[w2m[k * 16:(k + 1) * 16, :] for k in range(3)])

    agg2 = edge16(edges3, norm2d, xw2.reshape(N * R, L))

    out = _tc_call(
        _out_body, (GRID,),
        [aggblk(16), rowblk(16), rowblk(16), rowblk(16),
         full((16, L)), full((16, L)), full((16, L)), full((8, L))],
        rowblk(L),
        jax.ShapeDtypeStruct((N, L), f32),
    )(agg2[:, :N, :], h0, h1, h2,
      *[root2[k * 16:(k + 1) * 16, :] for k in range(3)], b2t)

    return out


# speedup vs baseline: 18.0232x; 1.0184x over previous
import functools

import jax
import jax.numpy as jnp
from jax import lax
from jax.experimental import pallas as pl
from jax.experimental.pallas import tpu as pltpu
from jax.experimental.pallas import tpu_sc as plsc

N = 50000
E = 800000
R = 8
D = 48
H = 48
L = 16

NTILES = 32
CHUNKS = 25
CHUNK = 1024
EP = NTILES * CHUNKS * CHUNK
EROWS = EP // 128
NK = 401408
NKROWS = NK // 128
NKT = NK // 16
NA = 50048
AROWS = NA // 16


def _xw1_body(emb_ref, w0_ref, w1_ref, w2_ref, o0_ref, o1_ref, o2_ref):
    x = emb_ref[...]
    o0_ref[...] = jnp.dot(x, w0_ref[...], preferred_element_type=jnp.float32)
    o1_ref[...] = jnp.dot(x, w1_ref[...], preferred_element_type=jnp.float32)
    o2_ref[...] = jnp.dot(x, w2_ref[...], preferred_element_type=jnp.float32)


def _inv_body(hist_ref, inv_ref):
    cnt = hist_ref[0] + hist_ref[1]
    row = lax.broadcasted_iota(jnp.int32, (NKROWS, 128), 0)
    inv_ref[...] = jnp.where(row < (N * R) // 128,
                             1.0 / jnp.maximum(cnt, 1.0), 0.0)


def _h1_body(a0_ref, a1_ref, a2_ref, emb_ref, r10_ref, r11_ref, r12_ref,
             b1_ref, w20_ref, w21_ref, w22_ref,
             h0_ref, h1_ref, h2_ref, xw2_ref):
    x = emb_ref[...]
    hs = []
    for k, (a_ref, r_ref) in enumerate(
            [(a0_ref, r10_ref), (a1_ref, r11_ref), (a2_ref, r12_ref)]):
        h = a_ref[0] + a_ref[1] + jnp.dot(
            x, r_ref[...], preferred_element_type=jnp.float32
        ) + b1_ref[0:1, k * 16:(k + 1) * 16]
        hs.append(jnp.maximum(h, 0.0))
    h0_ref[...] = hs[0]
    h1_ref[...] = hs[1]
    h2_ref[...] = hs[2]
    xw2_ref[...] = (
        jnp.dot(hs[0], w20_ref[...], preferred_element_type=jnp.float32)
        + jnp.dot(hs[1], w21_ref[...], preferred_element_type=jnp.float32)
        + jnp.dot(hs[2], w22_ref[...], preferred_element_type=jnp.float32))


def _out_body(agg2_ref, h0_ref, h1_ref, h2_ref, r20_ref, r21_ref, r22_ref,
              b2_ref, out_ref):
    y = (agg2_ref[0] + agg2_ref[1] + b2_ref[0:1, :]
         + jnp.dot(h0_ref[...], r20_ref[...],
                   preferred_element_type=jnp.float32)
         + jnp.dot(h1_ref[...], r21_ref[...],
                   preferred_element_type=jnp.float32)
         + jnp.dot(h2_ref[...], r22_ref[...],
                   preferred_element_type=jnp.float32))
    out_ref[...] = jax.nn.sigmoid(y)


def _sc_mesh():
    return plsc.VectorSubcoreMesh(core_axis_name="c", subcore_axis_name="s")


def _make_hist():
    mesh = _sc_mesh()

    @functools.partial(
        pl.kernel,
        out_type=jax.ShapeDtypeStruct((2 * NK,), jnp.float32),
        mesh=mesh,
        scratch_types=[
            pltpu.VMEM((8, 3, 128), jnp.int32),
            pltpu.VMEM((8, 128), jnp.int32),
            pltpu.VMEM((128,), jnp.float32),
            pltpu.VMEM((4096,), jnp.float32),
            pltpu.VMEM_SHARED((NK,), jnp.float32),
            pltpu.SemaphoreType.DMA,
            pltpu.SemaphoreType.DMA,
        ],
    )
    def hist(edges_hbm, out_hbm, ebuf, kidx, ones_v, zbuf, hist_sh,
             sem_l, sem_s):
        cid = lax.axis_index("c")
        sid = lax.axis_index("s")
        wid = cid * 16 + sid

        def zfill(i, _):
            zbuf[pl.ds(i * 16, 16)] = jnp.zeros((16,), jnp.float32)
            return _
        lax.fori_loop(0, 256, zfill, None)

        def ofill(i, _):
            ones_v[pl.ds(i * 16, 16)] = jnp.full((16,), 1.0, jnp.float32)
            return _
        lax.fori_loop(0, 8, ofill, None)

        zslot = sid * NKT
        for b in range(6):
            pltpu.sync_copy(zbuf.at[:],
                            hist_sh.at[pl.ds(zslot + b * 4096, 4096)])
        pltpu.sync_copy(zbuf.at[pl.ds(0, 512)],
                        hist_sh.at[pl.ds(zslot + 6 * 4096, 512)])
        plsc.subcore_barrier()

        def chunk_body(c, carry):
            base = (wid * CHUNKS + c) * 8
            pltpu.async_copy(edges_hbm.at[pl.ds(base, 8)], ebuf, sem_l).wait()
            for j in range(8):
                for g in range(8):
                    s = pl.ds(g * 16, 16)
                    kidx[j, s] = ebuf[j, 1, s] * R + ebuf[j, 2, s]
            adds = [pltpu.async_copy(ones_v, hist_sh.at[kidx.at[j]], sem_s,
                                     add=True) for j in range(8)]
            for a in adds:
                a.wait()
            return carry
        lax.fori_loop(0, CHUNKS, chunk_body, None)

        plsc.subcore_barrier()
        off = pl.multiple_of(cid * NK + sid * NKT, 128)
        pltpu.sync_copy(hist_sh.at[pl.ds(sid * NKT, NKT)],
                        out_hbm.at[pl.ds(off, NKT)])

    return hist


def _make_norm():
    mesh = _sc_mesh()

    @functools.partial(
        pl.kernel,
        out_type=jax.ShapeDtypeStruct((EROWS, 128), jnp.float32),
        mesh=mesh,
        scratch_types=[
            pltpu.VMEM((8, 3, 128), jnp.int32),
            pltpu.VMEM((8, 128), jnp.int32),
            pltpu.VMEM((8, 128), jnp.float32),
            pltpu.SemaphoreType.DMA,
        ],
    )
    def norm(edges_hbm, inv_hbm, out_hbm, ebuf, kidx, nbuf, sem):
        cid = lax.axis_index("c")
        sid = lax.axis_index("s")
        wid = cid * 16 + sid

        def chunk_body(c, carry):
            base = (wid * CHUNKS + c) * 8
            pltpu.async_copy(edges_hbm.at[pl.ds(base, 8)], ebuf, sem).wait()
            for j in range(8):
                for g in range(8):
                    s = pl.ds(g * 16, 16)
                    kidx[j, s] = ebuf[j, 1, s] * R + ebuf[j, 2, s]
            gs = [pltpu.async_copy(inv_hbm.at[kidx.at[j]], nbuf.at[j], sem)
                  for j in range(8)]
            for g_ in gs:
                g_.wait()
            pltpu.sync_copy(nbuf, out_hbm.at[pl.ds(base, 8)])
            return carry
        lax.fori_loop(0, CHUNKS, chunk_body, None)

    return norm


def _make_edge(W):
    mesh = _sc_mesh()
    nh = W // 16

    @functools.partial(
        pl.kernel,
        out_type=jax.ShapeDtypeStruct((2, NA, W), jnp.float32),
        mesh=mesh,
        scratch_types=[
            pltpu.VMEM((2, 8, 3, 128), jnp.int32),
            pltpu.VMEM((2, 8, 128), jnp.int32),
            pltpu.VMEM((2, 8, 128), jnp.int32),
            pltpu.VMEM((2, 8, 128), jnp.float32),
            pltpu.VMEM((2, 8, 128, W), jnp.float32),
            pltpu.VMEM((128, W), jnp.float32),
            pltpu.VMEM_SHARED((NA, W), jnp.float32),
            pltpu.SemaphoreType.DMA,
            pltpu.SemaphoreType.DMA,
            pltpu.SemaphoreType.DMA,
        ],
        compiler_params=pltpu.CompilerParams(use_tc_tiling_on_sc=False),
    )
    def edge(edges_hbm, norm_hbm, table_hbm, out_hbm,
             ebuf2, sidx2, didx2, nbuf2, rows2, zbuf, agg_sh,
             sem_l, sem_g, sem_s):
        cid = lax.axis_index("c")
        sid = lax.axis_index("s")
        wid = cid * 16 + sid

        def fire_lin(c, par):
            base = (wid * CHUNKS + c) * 8
            pltpu.async_copy(edges_hbm.at[pl.ds(base, 8)], ebuf2.at[par],
                             sem_l)
            pltpu.async_copy(norm_hbm.at[pl.ds(base, 8)], nbuf2.at[par],
                             sem_l)

        def wait_lin(par):
            pltpu.make_async_copy(edges_hbm.at[pl.ds(0, 8)], ebuf2.at[par],
                                  sem_l).wait()
            pltpu.make_async_copy(norm_hbm.at[pl.ds(0, 8)], nbuf2.at[par],
                                  sem_l).wait()

        def keys(par):
            for j in range(8):
                for g in range(8):
                    s = pl.ds(g * 16, 16)
                    sidx2[par, j, s] = (ebuf2[par, j, 0, s] * R
                                        + ebuf2[par, j, 2, s])
                    didx2[par, j, s] = ebuf2[par, j, 1, s]

        def fire_gathers(par):
            for j in range(8):
                pltpu.async_copy(table_hbm.at[sidx2.at[par, j]],
                                 rows2.at[par, j], sem_g)

        def process(par):
            for j in range(8):
                pltpu.make_async_copy(table_hbm.at[sidx2.at[par, j]],
                                      rows2.at[par, j], sem_g).wait()

                def scale(g, carry2):
                    nv = nbuf2[par, j, pl.ds(g * 16, 16)]
                    for i in range(16):
                        sc = nv[i]
                        e = g * 16 + i
                        for h in range(nh):
                            sl = pl.ds(h * 16, 16)
                            rows2[par, j, e, sl] = rows2[par, j, e, sl] * sc
                    return carry2
                lax.fori_loop(0, 8, scale, None)
                pltpu.async_copy(rows2.at[par, j], agg_sh.at[didx2.at[par, j]],
                                 sem_s, add=True)

        def drain_scatters(par):
            for j in range(8):
                pltpu.make_async_copy(rows2.at[par, j],
                                      agg_sh.at[didx2.at[par, j]],
                                      sem_s).wait()

        def zfill(i, carry):
            for h in range(nh):
                zbuf[i, pl.ds(h * 16, 16)] = jnp.zeros((16,), jnp.float32)
            return carry
        lax.fori_loop(0, 128, zfill, None)

        r0 = sid * AROWS
        for b in range(24):
            pltpu.sync_copy(zbuf.at[:], agg_sh.at[pl.ds(r0 + b * 128, 128)])
        pltpu.sync_copy(zbuf.at[pl.ds(0, 56)],
                        agg_sh.at[pl.ds(r0 + 24 * 128, 56)])
        plsc.subcore_barrier()

        fire_lin(0, 0)
        wait_lin(0)
        keys(0)
        fire_gathers(0)

        def body(c, carry):
            par = c & 1
            nxt = (c + 1) & 1
            fire_lin(c + 1, nxt)
            process(par)
            wait_lin(nxt)
            keys(nxt)
            fire_gathers(nxt)
            drain_scatters(par)
            return carry
        lax.fori_loop(0, CHUNKS - 1, body, None)

        last = (CHUNKS - 1) & 1
        process(last)
        drain_scatters(last)

        plsc.subcore_barrier()
        roff = pl.multiple_of(sid * AROWS, 8)
        pltpu.sync_copy(agg_sh.at[pl.ds(roff, AROWS)],
                        out_hbm.at[cid, pl.ds(roff, AROWS)])

    return edge


def _tc_call(body, grid, in_specs, out_specs, out_shape):
    return pl.pallas_call(body, grid=grid, in_specs=in_specs,
                          out_specs=out_specs, out_shape=out_shape)


def kernel(emb, weight1, root1, bias1, weight2, root2, bias2,
           edge_index, edge_type):
    f32 = jnp.float32
    src = edge_index[0]
    dst = edge_index[1]
    pad = EP - E
    src_p = jnp.concatenate([src, jnp.zeros((pad,), jnp.int32)])
    dst_p = jnp.concatenate([dst, jnp.full((pad,), N, jnp.int32)])
    typ_p = jnp.concatenate([edge_type, jnp.zeros((pad,), jnp.int32)])
    edges3 = jnp.stack([src_p.reshape(EROWS, 128),
                        dst_p.reshape(EROWS, 128),
                        typ_p.reshape(EROWS, 128)], axis=1)

    w1s = [weight1[:, :, k * 16:(k + 1) * 16].transpose(1, 0, 2)
           .reshape(D, R * 16) for k in range(3)]
    w2m = weight2.transpose(1, 0, 2).reshape(H, R * L)
    b1t = jnp.tile(bias1.reshape(1, H), (8, 1))
    b2t = jnp.tile(bias2.reshape(1, L), (8, 1))

    BN = 1000
    GRID = N // BN

    full = lambda shp: pl.BlockSpec(shp, lambda i: (0,) * len(shp))
    rowblk = lambda w: pl.BlockSpec((BN, w), lambda i: (i, 0))
    aggblk = lambda w: pl.BlockSpec((2, BN, w), lambda i: (0, i, 0))

    xws = _tc_call(
        _xw1_body, (GRID,),
        [rowblk(D)] + [full((D, R * 16))] * 3,
        [rowblk(R * 16)] * 3,
        [jax.ShapeDtypeStruct((N, R * 16), f32)] * 3,
    )(emb, *w1s)
    xws = [x.reshape(N * R, 16) for x in xws]

    hist = _make_hist()(edges3)

    inv = _tc_call(
        _inv_body, (1,),
        [pl.BlockSpec((2, NKROWS, 128), lambda i: (0, 0, 0))],
        pl.BlockSpec((NKROWS, 128), lambda i: (0, 0)),
        jax.ShapeDtypeStruct((NKROWS, 128), f32),
    )(hist.reshape(2, NKROWS, 128)).reshape(NK)

    norm2d = _make_norm()(edges3, inv)

    edge16 = _make_edge(16)
    aggs = [edge16(edges3, norm2d, x) for x in xws]

    h0, h1, h2, xw2 = _tc_call(
        _h1_body, (GRID,),
        [aggblk(16)] * 3 + [rowblk(D)] + [full((D, 16))] * 3
        + [full((8, H))] + [full((16, R * L))] * 3,
        [rowblk(16)] * 3 + [rowblk(R * L)],
        [jax.ShapeDtypeStruct((N, 16), f32)] * 3
        + [jax.ShapeDtypeStruct((N, R * L), f32)],
    )(*[a[:, :N, :] for a in aggs], emb,
      *[root1[:, k * 16:(k + 1) * 16] for k in range(3)], b1t,
      *[w2m[k * 16:(k + 1) * 16, :] for k in range(3)])

    agg2 = edge16(edges3, norm2d, xw2.reshape(N * R, L))

    out = _tc_call(
        _out_body, (GRID,),
        [aggblk(16), rowblk(16), rowblk(16), rowblk(16),
         full((16, L)), full((16, L)), full((16, L)), full((8, L))],
        rowblk(L),
        jax.ShapeDtypeStruct((N, L), f32),
    )(agg2[:, :N, :], h0, h1, h2,
      *[root2[k * 16:(k + 1) * 16, :] for k in range(3)], b2t)

    return out


# speedup vs baseline: 18.9899x; 1.0536x over previous
import functools

import jax
import jax.numpy as jnp
from jax import lax
from jax.experimental import pallas as pl
from jax.experimental.pallas import tpu as pltpu
from jax.experimental.pallas import tpu_sc as plsc

N = 50000
E = 800000
R = 8
D = 48
H = 48
L = 16

NTILES = 32
CHUNKS = 25
SLOW_CID = 1
FAST_CHUNKS = 33
SLOW_CHUNKS = 17
CHUNK = 1024
EP = NTILES * CHUNKS * CHUNK
EROWS = EP // 128
NK = 401408
NKROWS = NK // 128
NKT = NK // 16
NA = 50048
AROWS = NA // 16


def _xw1_body(emb_ref, w0_ref, w1_ref, w2_ref, o0_ref, o1_ref, o2_ref):
    x = emb_ref[...]
    o0_ref[...] = jnp.dot(x, w0_ref[...], preferred_element_type=jnp.float32)
    o1_ref[...] = jnp.dot(x, w1_ref[...], preferred_element_type=jnp.float32)
    o2_ref[...] = jnp.dot(x, w2_ref[...], preferred_element_type=jnp.float32)


def _inv_body(hist_ref, inv_ref):
    cnt = hist_ref[0] + hist_ref[1]
    row = lax.broadcasted_iota(jnp.int32, (NKROWS, 128), 0)
    inv_ref[...] = jnp.where(row < (N * R) // 128,
                             1.0 / jnp.maximum(cnt, 1.0), 0.0)


def _h1_body(a0_ref, a1_ref, a2_ref, emb_ref, r10_ref, r11_ref, r12_ref,
             b1_ref, w20_ref, w21_ref, w22_ref,
             h0_ref, h1_ref, h2_ref, xw2_ref):
    x = emb_ref[...]
    hs = []
    for k, (a_ref, r_ref) in enumerate(
            [(a0_ref, r10_ref), (a1_ref, r11_ref), (a2_ref, r12_ref)]):
        h = a_ref[0] + a_ref[1] + jnp.dot(
            x, r_ref[...], preferred_element_type=jnp.float32
        ) + b1_ref[0:1, k * 16:(k + 1) * 16]
        hs.append(jnp.maximum(h, 0.0))
    h0_ref[...] = hs[0]
    h1_ref[...] = hs[1]
    h2_ref[...] = hs[2]
    xw2_ref[...] = (
        jnp.dot(hs[0], w20_ref[...], preferred_element_type=jnp.float32)
        + jnp.dot(hs[1], w21_ref[...], preferred_element_type=jnp.float32)
        + jnp.dot(hs[2], w22_ref[...], preferred_element_type=jnp.float32))


def _out_body(agg2_ref, h0_ref, h1_ref, h2_ref, r20_ref, r21_ref, r22_ref,
              b2_ref, out_ref):
    y = (agg2_ref[0] + agg2_ref[1] + b2_ref[0:1, :]
         + jnp.dot(h0_ref[...], r20_ref[...],
                   preferred_element_type=jnp.float32)
         + jnp.dot(h1_ref[...], r21_ref[...],
                   preferred_element_type=jnp.float32)
         + jnp.dot(h2_ref[...], r22_ref[...],
                   preferred_element_type=jnp.float32))
    out_ref[...] = jax.nn.sigmoid(y)


def _sc_mesh():
    return plsc.VectorSubcoreMesh(core_axis_name="c", subcore_axis_name="s")


def _make_hist():
    mesh = _sc_mesh()

    @functools.partial(
        pl.kernel,
        out_type=jax.ShapeDtypeStruct((2 * NK,), jnp.float32),
        mesh=mesh,
        scratch_types=[
            pltpu.VMEM((8, 3, 128), jnp.int32),
            pltpu.VMEM((8, 128), jnp.int32),
            pltpu.VMEM((128,), jnp.float32),
            pltpu.VMEM((4096,), jnp.float32),
            pltpu.VMEM_SHARED((NK,), jnp.float32),
            pltpu.SemaphoreType.DMA,
            pltpu.SemaphoreType.DMA,
        ],
    )
    def hist(edges_hbm, out_hbm, ebuf, kidx, ones_v, zbuf, hist_sh,
             sem_l, sem_s):
        cid = lax.axis_index("c")
        sid = lax.axis_index("s")
        wid = cid * 16 + sid

        def zfill(i, _):
            zbuf[pl.ds(i * 16, 16)] = jnp.zeros((16,), jnp.float32)
            return _
        lax.fori_loop(0, 256, zfill, None)

        def ofill(i, _):
            ones_v[pl.ds(i * 16, 16)] = jnp.full((16,), 1.0, jnp.float32)
            return _
        lax.fori_loop(0, 8, ofill, None)

        zslot = sid * NKT
        for b in range(6):
            pltpu.sync_copy(zbuf.at[:],
                            hist_sh.at[pl.ds(zslot + b * 4096, 4096)])
        pltpu.sync_copy(zbuf.at[pl.ds(0, 512)],
                        hist_sh.at[pl.ds(zslot + 6 * 4096, 512)])
        plsc.subcore_barrier()

        def chunk_body(c, carry):
            base = (wid * CHUNKS + c) * 8
            pltpu.async_copy(edges_hbm.at[pl.ds(base, 8)], ebuf, sem_l).wait()
            for j in range(8):
                for g in range(8):
                    s = pl.ds(g * 16, 16)
                    kidx[j, s] = ebuf[j, 1, s] * R + ebuf[j, 2, s]
            adds = [pltpu.async_copy(ones_v, hist_sh.at[kidx.at[j]], sem_s,
                                     add=True) for j in range(8)]
            for a in adds:
                a.wait()
            return carry
        lax.fori_loop(0, CHUNKS, chunk_body, None)

        plsc.subcore_barrier()
        off = pl.multiple_of(cid * NK + sid * NKT, 128)
        pltpu.sync_copy(hist_sh.at[pl.ds(sid * NKT, NKT)],
                        out_hbm.at[pl.ds(off, NKT)])

    return hist


def _make_norm():
    mesh = _sc_mesh()

    @functools.partial(
        pl.kernel,
        out_type=jax.ShapeDtypeStruct((EROWS, 128), jnp.float32),
        mesh=mesh,
        scratch_types=[
            pltpu.VMEM((8, 3, 128), jnp.int32),
            pltpu.VMEM((8, 128), jnp.int32),
            pltpu.VMEM((8, 128), jnp.float32),
            pltpu.SemaphoreType.DMA,
        ],
    )
    def norm(edges_hbm, inv_hbm, out_hbm, ebuf, kidx, nbuf, sem):
        cid = lax.axis_index("c")
        sid = lax.axis_index("s")
        slow = cid == SLOW_CID
        nc = jnp.where(slow, SLOW_CHUNKS, FAST_CHUNKS)
        c0 = jnp.where(slow, 16 * FAST_CHUNKS + sid * SLOW_CHUNKS,
                       sid * FAST_CHUNKS)

        def chunk_body(c, carry):
            base = (c0 + c) * 8
            pltpu.async_copy(edges_hbm.at[pl.ds(base, 8)], ebuf, sem).wait()
            for j in range(8):
                for g in range(8):
                    s = pl.ds(g * 16, 16)
                    kidx[j, s] = ebuf[j, 1, s] * R + ebuf[j, 2, s]
            gs = [pltpu.async_copy(inv_hbm.at[kidx.at[j]], nbuf.at[j], sem)
                  for j in range(8)]
            for g_ in gs:
                g_.wait()
            pltpu.sync_copy(nbuf, out_hbm.at[pl.ds(base, 8)])
            return carry
        lax.fori_loop(0, nc, chunk_body, None)

    return norm


def _make_edge(W):
    mesh = _sc_mesh()
    nh = W // 16

    @functools.partial(
        pl.kernel,
        out_type=jax.ShapeDtypeStruct((2, NA, W), jnp.float32),
        mesh=mesh,
        scratch_types=[
            pltpu.VMEM((2, 8, 3, 128), jnp.int32),
            pltpu.VMEM((2, 8, 128), jnp.int32),
            pltpu.VMEM((2, 8, 128), jnp.int32),
            pltpu.VMEM((2, 8, 128), jnp.float32),
            pltpu.VMEM((2, 8, 128, W), jnp.float32),
            pltpu.VMEM((128, W), jnp.float32),
            pltpu.VMEM_SHARED((NA, W), jnp.float32),
            pltpu.SemaphoreType.DMA,
            pltpu.SemaphoreType.DMA,
            pltpu.SemaphoreType.DMA,
        ],
        compiler_params=pltpu.CompilerParams(use_tc_tiling_on_sc=False),
    )
    def edge(edges_hbm, norm_hbm, table_hbm, out_hbm,
             ebuf2, sidx2, didx2, nbuf2, rows2, zbuf, agg_sh,
             sem_l, sem_g, sem_s):
        cid = lax.axis_index("c")
        sid = lax.axis_index("s")
        slow = cid == SLOW_CID
        nc = jnp.where(slow, SLOW_CHUNKS, FAST_CHUNKS)
        c0 = jnp.where(slow, 16 * FAST_CHUNKS + sid * SLOW_CHUNKS,
                       sid * FAST_CHUNKS)

        def fire_lin(c, par):
            base = (c0 + c) * 8
            pltpu.async_copy(edges_hbm.at[pl.ds(base, 8)], ebuf2.at[par],
                             sem_l)
            pltpu.async_copy(norm_hbm.at[pl.ds(base, 8)], nbuf2.at[par],
                             sem_l)

        def wait_lin(par):
            pltpu.make_async_copy(edges_hbm.at[pl.ds(0, 8)], ebuf2.at[par],
                                  sem_l).wait()
            pltpu.make_async_copy(norm_hbm.at[pl.ds(0, 8)], nbuf2.at[par],
                                  sem_l).wait()

        def keys(par):
            for j in range(8):
                for g in range(8):
                    s = pl.ds(g * 16, 16)
                    sidx2[par, j, s] = (ebuf2[par, j, 0, s] * R
                                        + ebuf2[par, j, 2, s])
                    didx2[par, j, s] = ebuf2[par, j, 1, s]

        def fire_gathers(par):
            for j in range(8):
                pltpu.async_copy(table_hbm.at[sidx2.at[par, j]],
                                 rows2.at[par, j], sem_g)

        def process(par):
            for j in range(8):
                pltpu.make_async_copy(table_hbm.at[sidx2.at[par, j]],
                                      rows2.at[par, j], sem_g).wait()

                def scale(g, carry2):
                    nv = nbuf2[par, j, pl.ds(g * 16, 16)]
                    for i in range(16):
                        sc = nv[i]
                        e = g * 16 + i
                        for h in range(nh):
                            sl = pl.ds(h * 16, 16)
                            rows2[par, j, e, sl] = rows2[par, j, e, sl] * sc
                    return carry2
                lax.fori_loop(0, 8, scale, None)
                pltpu.async_copy(rows2.at[par, j], agg_sh.at[didx2.at[par, j]],
                                 sem_s, add=True)

        def drain_scatters(par):
            for j in range(8):
                pltpu.make_async_copy(rows2.at[par, j],
                                      agg_sh.at[didx2.at[par, j]],
                                      sem_s).wait()

        def zfill(i, carry):
            for h in range(nh):
                zbuf[i, pl.ds(h * 16, 16)] = jnp.zeros((16,), jnp.float32)
            return carry
        lax.fori_loop(0, 128, zfill, None)

        r0 = sid * AROWS
        for b in range(24):
            pltpu.sync_copy(zbuf.at[:], agg_sh.at[pl.ds(r0 + b * 128, 128)])
        pltpu.sync_copy(zbuf.at[pl.ds(0, 56)],
                        agg_sh.at[pl.ds(r0 + 24 * 128, 56)])
        plsc.subcore_barrier()

        fire_lin(0, 0)
        wait_lin(0)
        keys(0)
        fire_gathers(0)

        def body(c, carry):
            par = c & 1
            nxt = (c + 1) & 1
            fire_lin(c + 1, nxt)
            process(par)
            wait_lin(nxt)
            keys(nxt)
            fire_gathers(nxt)
            drain_scatters(par)
            return carry
        lax.fori_loop(0, nc - 1, body, None)

        last = (nc - 1) & 1
        process(last)
        drain_scatters(last)

        plsc.subcore_barrier()
        roff = pl.multiple_of(sid * AROWS, 8)
        pltpu.sync_copy(agg_sh.at[pl.ds(roff, AROWS)],
                        out_hbm.at[cid, pl.ds(roff, AROWS)])

    return edge


def _tc_call(body, grid, in_specs, out_specs, out_shape):
    return pl.pallas_call(body, grid=grid, in_specs=in_specs,
                          out_specs=out_specs, out_shape=out_shape)


def kernel(emb, weight1, root1, bias1, weight2, root2, bias2,
           edge_index, edge_type):
    f32 = jnp.float32
    src = edge_index[0]
    dst = edge_index[1]
    pad = EP - E
    src_p = jnp.concatenate([src, jnp.zeros((pad,), jnp.int32)])
    dst_p = jnp.concatenate([dst, jnp.full((pad,), N, jnp.int32)])
    typ_p = jnp.concatenate([edge_type, jnp.zeros((pad,), jnp.int32)])
    edges3 = jnp.stack([src_p.reshape(EROWS, 128),
                        dst_p.reshape(EROWS, 128),
                        typ_p.reshape(EROWS, 128)], axis=1)

    w1s = [weight1[:, :, k * 16:(k + 1) * 16].transpose(1, 0, 2)
           .reshape(D, R * 16) for k in range(3)]
    w2m = weight2.transpose(1, 0, 2).reshape(H, R * L)
    b1t = jnp.tile(bias1.reshape(1, H), (8, 1))
    b2t = jnp.tile(bias2.reshape(1, L), (8, 1))

    BN = 1000
    GRID = N // BN

    full = lambda shp: pl.BlockSpec(shp, lambda i: (0,) * len(shp))
    rowblk = lambda w: pl.BlockSpec((BN, w), lambda i: (i, 0))
    aggblk = lambda w: pl.BlockSpec((2, BN, w), lambda i: (0, i, 0))

    xws = _tc_call(
        _xw1_body, (GRID,),
        [rowblk(D)] + [full((D, R * 16))] * 3,
        [rowblk(R * 16)] * 3,
        [jax.ShapeDtypeStruct((N, R * 16), f32)] * 3,
    )(emb, *w1s)
    xws = [x.reshape(N * R, 16) for x in xws]

    hist = _make_hist()(edges3)

    inv = _tc_call(
        _inv_body, (1,),
        [pl.BlockSpec((2, NKROWS, 128), lambda i: (0, 0, 0))],
        pl.BlockSpec((NKROWS, 128), lambda i: (0, 0)),
        jax.ShapeDtypeStruct((NKROWS, 128), f32),
    )(hist.reshape(2, NKROWS, 128)).reshape(NK)

    norm2d = _make_norm()(edges3, inv)

    edge16 = _make_edge(16)
    aggs = [edge16(edges3, norm2d, x) for x in xws]

    h0, h1, h2, xw2 = _tc_call(
        _h1_body, (GRID,),
        [aggblk(16)] * 3 + [rowblk(D)] + [full((D, 16))] * 3
        + [full((8, H))] + [full((16, R * L))] * 3,
        [rowblk(16)] * 3 + [rowblk(R * L)],
        [jax.ShapeDtypeStruct((N, 16), f32)] * 3
        + [jax.ShapeDtypeStruct((N, R * L), f32)],
    )(*[a[:, :N, :] for a in aggs], emb,
      *[root1[:, k * 16:(k + 1) * 16] for k in range(3)], b1t,
      *[w2m[k * 16:(k + 1) * 16, :] for k in range(3)])

    agg2 = edge16(edges3, norm2d, xw2.reshape(N * R, L))

    out = _tc_call(
        _out_body, (GRID,),
        [aggblk(16), rowblk(16), rowblk(16), rowblk(16),
         full((16, L)), full((16, L)), full((16, L)), full((8, L))],
        rowblk(L),
        jax.ShapeDtypeStruct((N, L), f32),
    )(agg2[:, :N, :], h0, h1, h2,
      *[root2[k * 16:(k + 1) * 16, :] for k in range(3)], b2t)

    return out


# speedup vs baseline: 20.3701x; 1.0727x over previous
import functools

import jax
import jax.numpy as jnp
from jax import lax
from jax.experimental import pallas as pl
from jax.experimental.pallas import tpu as pltpu
from jax.experimental.pallas import tpu_sc as plsc

N = 50000
E = 800000
R = 8
D = 48
H = 48
L = 16

NTILES = 32
CHUNKS = 25
SLOW_CID = 1
FAST_CHUNKS = 33
SLOW_CHUNKS = 17
CHUNK = 1024
EP = NTILES * CHUNKS * CHUNK
EROWS = EP // 128
NK = 401408
NKROWS = NK // 128
NKT = NK // 16
NA = 50048
AROWS = NA // 16


def _xw1_body(emb_ref, w_ref, o0_ref, o1_ref, o2_ref):
    x = emb_ref[...]
    y = jnp.dot(x, w_ref[...], preferred_element_type=jnp.float32)
    o0_ref[...] = y[:, :128]
    o1_ref[...] = y[:, 128:256]
    o2_ref[...] = y[:, 256:]


def _inv_body(hist_ref, inv_ref):
    cnt = hist_ref[0] + hist_ref[1]
    row = lax.broadcasted_iota(jnp.int32, (NKROWS, 128), 0)
    inv_ref[...] = jnp.where(row < (N * R) // 128,
                             1.0 / jnp.maximum(cnt, 1.0), 0.0)


def _h1_body(a0_ref, a1_ref, a2_ref, emb_ref, r1_ref, b1_ref, w2_ref,
             h_ref, xw2_ref):
    x = emb_ref[...]
    acat = jnp.concatenate(
        [a0_ref[0] + a0_ref[1], a1_ref[0] + a1_ref[1], a2_ref[0] + a2_ref[1]],
        axis=-1)
    h = acat + jnp.dot(x, r1_ref[...], preferred_element_type=jnp.float32)
    h = jnp.maximum(h + b1_ref[0:1, :], 0.0)
    h_ref[...] = h
    xw2_ref[...] = jnp.dot(h, w2_ref[...], preferred_element_type=jnp.float32)


def _out_body(agg2_ref, h_ref, r2_ref, b2_ref, out_ref):
    y = (agg2_ref[0] + agg2_ref[1] + b2_ref[0:1, :]
         + jnp.dot(h_ref[...], r2_ref[...],
                   preferred_element_type=jnp.float32))
    out_ref[...] = jax.nn.sigmoid(y)


def _sc_mesh():
    return plsc.VectorSubcoreMesh(core_axis_name="c", subcore_axis_name="s")


def _make_hist():
    mesh = _sc_mesh()

    @functools.partial(
        pl.kernel,
        out_type=jax.ShapeDtypeStruct((2 * NK,), jnp.float32),
        mesh=mesh,
        scratch_types=[
            pltpu.VMEM((8, 3, 128), jnp.int32),
            pltpu.VMEM((8, 128), jnp.int32),
            pltpu.VMEM((128,), jnp.float32),
            pltpu.VMEM((4096,), jnp.float32),
            pltpu.VMEM_SHARED((NK,), jnp.float32),
            pltpu.SemaphoreType.DMA,
            pltpu.SemaphoreType.DMA,
        ],
    )
    def hist(edges_hbm, out_hbm, ebuf, kidx, ones_v, zbuf, hist_sh,
             sem_l, sem_s):
        cid = lax.axis_index("c")
        sid = lax.axis_index("s")
        wid = cid * 16 + sid

        def zfill(i, _):
            zbuf[pl.ds(i * 16, 16)] = jnp.zeros((16,), jnp.float32)
            return _
        lax.fori_loop(0, 256, zfill, None)

        def ofill(i, _):
            ones_v[pl.ds(i * 16, 16)] = jnp.full((16,), 1.0, jnp.float32)
            return _
        lax.fori_loop(0, 8, ofill, None)

        zslot = sid * NKT
        for b in range(6):
            pltpu.sync_copy(zbuf.at[:],
                            hist_sh.at[pl.ds(zslot + b * 4096, 4096)])
        pltpu.sync_copy(zbuf.at[pl.ds(0, 512)],
                        hist_sh.at[pl.ds(zslot + 6 * 4096, 512)])
        plsc.subcore_barrier()

        def chunk_body(c, carry):
            base = (wid * CHUNKS + c) * 8
            pltpu.async_copy(edges_hbm.at[pl.ds(base, 8)], ebuf, sem_l).wait()
            for j in range(8):
                for g in range(8):
                    s = pl.ds(g * 16, 16)
                    kidx[j, s] = ebuf[j, 1, s] * R + ebuf[j, 2, s]
            adds = [pltpu.async_copy(ones_v, hist_sh.at[kidx.at[j]], sem_s,
                                     add=True) for j in range(8)]
            for a in adds:
                a.wait()
            return carry
        lax.fori_loop(0, CHUNKS, chunk_body, None)

        plsc.subcore_barrier()
        off = pl.multiple_of(cid * NK + sid * NKT, 128)
        pltpu.sync_copy(hist_sh.at[pl.ds(sid * NKT, NKT)],
                        out_hbm.at[pl.ds(off, NKT)])

    return hist


def _make_norm():
    mesh = _sc_mesh()

    @functools.partial(
        pl.kernel,
        out_type=jax.ShapeDtypeStruct((EROWS, 128), jnp.float32),
        mesh=mesh,
        scratch_types=[
            pltpu.VMEM((8, 3, 128), jnp.int32),
            pltpu.VMEM((8, 128), jnp.int32),
            pltpu.VMEM((8, 128), jnp.float32),
            pltpu.SemaphoreType.DMA,
        ],
    )
    def norm(edges_hbm, inv_hbm, out_hbm, ebuf, kidx, nbuf, sem):
        cid = lax.axis_index("c")
        sid = lax.axis_index("s")
        slow = cid == SLOW_CID
        nc = jnp.where(slow, SLOW_CHUNKS, FAST_CHUNKS)
        c0 = jnp.where(slow, 16 * FAST_CHUNKS + sid * SLOW_CHUNKS,
                       sid * FAST_CHUNKS)

        def chunk_body(c, carry):
            base = (c0 + c) * 8
            pltpu.async_copy(edges_hbm.at[pl.ds(base, 8)], ebuf, sem).wait()
            for j in range(8):
                for g in range(8):
                    s = pl.ds(g * 16, 16)
                    kidx[j, s] = ebuf[j, 1, s] * R + ebuf[j, 2, s]
            gs = [pltpu.async_copy(inv_hbm.at[kidx.at[j]], nbuf.at[j], sem)
                  for j in range(8)]
            for g_ in gs:
                g_.wait()
            pltpu.sync_copy(nbuf, out_hbm.at[pl.ds(base, 8)])
            return carry
        lax.fori_loop(0, nc, chunk_body, None)

    return norm


def _make_edge(W):
    mesh = _sc_mesh()
    nh = W // 16

    @functools.partial(
        pl.kernel,
        out_type=jax.ShapeDtypeStruct((2, NA, W), jnp.float32),
        mesh=mesh,
        scratch_types=[
            pltpu.VMEM((2, 8, 3, 128), jnp.int32),
            pltpu.VMEM((2, 8, 128), jnp.int32),
            pltpu.VMEM((2, 8, 128), jnp.int32),
            pltpu.VMEM((2, 8, 128), jnp.float32),
            pltpu.VMEM((2, 8, 128, W), jnp.float32),
            pltpu.VMEM((128, W), jnp.float32),
            pltpu.VMEM_SHARED((NA, W), jnp.float32),
            pltpu.SemaphoreType.DMA,
            pltpu.SemaphoreType.DMA,
            pltpu.SemaphoreType.DMA,
        ],
        compiler_params=pltpu.CompilerParams(use_tc_tiling_on_sc=False),
    )
    def edge(edges_hbm, norm_hbm, table_hbm, out_hbm,
             ebuf2, sidx2, didx2, nbuf2, rows2, zbuf, agg_sh,
             sem_l, sem_g, sem_s):
        cid = lax.axis_index("c")
        sid = lax.axis_index("s")
        slow = cid == SLOW_CID
        nc = jnp.where(slow, SLOW_CHUNKS, FAST_CHUNKS)
        c0 = jnp.where(slow, 16 * FAST_CHUNKS + sid * SLOW_CHUNKS,
                       sid * FAST_CHUNKS)

        def fire_lin(c, par):
            base = (c0 + c) * 8
            pltpu.async_copy(edges_hbm.at[pl.ds(base, 8)], ebuf2.at[par],
                             sem_l)
            pltpu.async_copy(norm_hbm.at[pl.ds(base, 8)], nbuf2.at[par],
                             sem_l)

        def wait_lin(par):
            pltpu.make_async_copy(edges_hbm.at[pl.ds(0, 8)], ebuf2.at[par],
                                  sem_l).wait()
            pltpu.make_async_copy(norm_hbm.at[pl.ds(0, 8)], nbuf2.at[par],
                                  sem_l).wait()

        def keys(par):
            for j in range(8):
                for g in range(8):
                    s = pl.ds(g * 16, 16)
                    sidx2[par, j, s] = (ebuf2[par, j, 0, s] * R
                                        + ebuf2[par, j, 2, s])
                    didx2[par, j, s] = ebuf2[par, j, 1, s]

        def fire_gathers(par):
            for j in range(8):
                pltpu.async_copy(table_hbm.at[sidx2.at[par, j]],
                                 rows2.at[par, j], sem_g)

        def process(par):
            for j in range(8):
                pltpu.make_async_copy(table_hbm.at[sidx2.at[par, j]],
                                      rows2.at[par, j], sem_g).wait()

                def scale(g, carry2):
                    nv = nbuf2[par, j, pl.ds(g * 16, 16)]
                    for i in range(16):
                        sc = nv[i]
                        e = g * 16 + i
                        for h in range(nh):
                            sl = pl.ds(h * 16, 16)
                            rows2[par, j, e, sl] = rows2[par, j, e, sl] * sc
                    return carry2
                lax.fori_loop(0, 8, scale, None)
                pltpu.async_copy(rows2.at[par, j], agg_sh.at[didx2.at[par, j]],
                                 sem_s, add=True)

        def drain_scatters(par):
            for j in range(8):
                pltpu.make_async_copy(rows2.at[par, j],
                                      agg_sh.at[didx2.at[par, j]],
                                      sem_s).wait()

        def zfill(i, carry):
            for h in range(nh):
                zbuf[i, pl.ds(h * 16, 16)] = jnp.zeros((16,), jnp.float32)
            return carry
        lax.fori_loop(0, 128, zfill, None)

        r0 = sid * AROWS
        for b in range(24):
            pltpu.sync_copy(zbuf.at[:], agg_sh.at[pl.ds(r0 + b * 128, 128)])
        pltpu.sync_copy(zbuf.at[pl.ds(0, 56)],
                        agg_sh.at[pl.ds(r0 + 24 * 128, 56)])
        plsc.subcore_barrier()

        fire_lin(0, 0)
        wait_lin(0)
        keys(0)
        fire_gathers(0)

        def body(c, carry):
            par = c & 1
            nxt = (c + 1) & 1
            fire_lin(c + 1, nxt)
            process(par)
            wait_lin(nxt)
            keys(nxt)
            fire_gathers(nxt)
            drain_scatters(par)
            return carry
        lax.fori_loop(0, nc - 1, body, None)

        last = (nc - 1) & 1
        process(last)
        drain_scatters(last)

        plsc.subcore_barrier()
        roff = pl.multiple_of(sid * AROWS, 8)
        pltpu.sync_copy(agg_sh.at[pl.ds(roff, AROWS)],
                        out_hbm.at[cid, pl.ds(roff, AROWS)])

    return edge


def _tc_call(body, grid, in_specs, out_specs, out_shape):
    return pl.pallas_call(body, grid=grid, in_specs=in_specs,
                          out_specs=out_specs, out_shape=out_shape)


def kernel(emb, weight1, root1, bias1, weight2, root2, bias2,
           edge_index, edge_type):
    f32 = jnp.float32
    src = edge_index[0]
    dst = edge_index[1]
    pad = EP - E
    src_p = jnp.concatenate([src, jnp.zeros((pad,), jnp.int32)])
    dst_p = jnp.concatenate([dst, jnp.full((pad,), N, jnp.int32)])
    typ_p = jnp.concatenate([edge_type, jnp.zeros((pad,), jnp.int32)])
    edges3 = jnp.stack([src_p.reshape(EROWS, 128),
                        dst_p.reshape(EROWS, 128),
                        typ_p.reshape(EROWS, 128)], axis=1)

    w1s = [weight1[:, :, k * 16:(k + 1) * 16].transpose(1, 0, 2)
           .reshape(D, R * 16) for k in range(3)]
    w2m = weight2.transpose(1, 0, 2).reshape(H, R * L)
    b1t = jnp.tile(bias1.reshape(1, H), (8, 1))
    b2t = jnp.tile(bias2.reshape(1, L), (8, 1))

    BN = 2000
    GRID = N // BN

    full = lambda shp: pl.BlockSpec(shp, lambda i: (0,) * len(shp))
    rowblk = lambda w: pl.BlockSpec((BN, w), lambda i: (i, 0))
    aggblk = lambda w: pl.BlockSpec((2, BN, w), lambda i: (0, i, 0))

    xws = _tc_call(
        _xw1_body, (GRID,),
        [rowblk(D), full((D, 3 * R * 16))],
        [rowblk(R * 16)] * 3,
        [jax.ShapeDtypeStruct((N, R * 16), f32)] * 3,
    )(emb, jnp.concatenate(w1s, axis=1))
    xws = [x.reshape(N * R, 16) for x in xws]

    hist = _make_hist()(edges3)

    inv = _tc_call(
        _inv_body, (1,),
        [pl.BlockSpec((2, NKROWS, 128), lambda i: (0, 0, 0))],
        pl.BlockSpec((NKROWS, 128), lambda i: (0, 0)),
        jax.ShapeDtypeStruct((NKROWS, 128), f32),
    )(hist.reshape(2, NKROWS, 128)).reshape(NK)

    norm2d = _make_norm()(edges3, inv)

    edge16 = _make_edge(16)
    aggs = [edge16(edges3, norm2d, x) for x in xws]

    h1f, xw2 = _tc_call(
        _h1_body, (GRID,),
        [aggblk(16)] * 3 + [rowblk(D), full((D, H)), full((8, H)),
                            full((H, R * L))],
        [rowblk(H), rowblk(R * L)],
        [jax.ShapeDtypeStruct((N, H), f32),
         jax.ShapeDtypeStruct((N, R * L), f32)],
    )(*[a[:, :N, :] for a in aggs], emb, root1, b1t, w2m)

    agg2 = edge16(edges3, norm2d, xw2.reshape(N * R, L))

    out = _tc_call(
        _out_body, (GRID,),
        [aggblk(16), rowblk(H), full((H, L)), full((8, L))],
        rowblk(L),
        jax.ShapeDtypeStruct((N, L), f32),
    )(agg2[:, :N, :], h1f, root2, b2t)

    return out
